# baseline (speedup 1.0000x reference)
import functools
import math

import numpy as np

import jax
import jax.numpy as jnp
from jax import lax
from jax.experimental import pallas as pl
from jax.experimental.pallas import tpu as pltpu
from jax.experimental.pallas import tpu_sc as plsc

N = 10000
E = 320000
D = 128
DW = D // 2

NC = 2
NS = 16
NW = NC * NS

CH = 64
IG = 20
IGN = 8
NCHUNK = IG * IGN
EPW = NCHUNK * CH
E_PAD = NW * EPW
NBG = 3
NBS = 2

N_PAD = 10240
RS = N_PAD // NS
ZR = 16

_COL_SRC = np.empty((D,), np.int32)
for _j in range(D // 32):
    for _k in range(16):
        _COL_SRC[32 * _j + 2 * _k] = 32 * _j + _k
        _COL_SRC[32 * _j + 2 * _k + 1] = 32 * _j + 16 + _k


def _spmm_body(src_hbm, dst_hbm, vals_hbm, h_hbm, out0_hbm, out1_hbm,
               src_v, dst_v, vals_v, rows_bf, rows_f, zbuf, acc_sh,
               gsems, ssems):
    c = lax.axis_index("c")
    s = lax.axis_index("s")
    w = c * NS + s
    row0 = s * RS

    def _zero_row(i, carry):
        for j in range(D // 16):
            zbuf[i, pl.ds(j * 16, 16)] = jnp.zeros((16,), jnp.float32)
        return carry

    lax.fori_loop(0, ZR, _zero_row, 0)
    for k in range(RS // ZR):
        pltpu.sync_copy(zbuf, acc_sh.at[pl.ds(row0 + k * ZR, ZR)])
    plsc.subcore_barrier()

    hi_mask = jnp.full((16,), -65536, jnp.int32)

    def _outer(og, carry):
        pltpu.sync_copy(src_hbm.at[w, og], src_v)
        pltpu.sync_copy(dst_hbm.at[w, og], dst_v)
        pltpu.sync_copy(vals_hbm.at[w, og], vals_v)

        pltpu.async_copy(h_hbm.at[src_v.at[0]], rows_bf.at[0], gsems.at[0])
        pltpu.async_copy(h_hbm.at[src_v.at[1]], rows_bf.at[1], gsems.at[1])

        def _chunk(g, cc2):
            gb = lax.rem(g, NBG)
            sb = lax.rem(g, NBS)

            @pl.when(g >= NBS)
            def _():
                pltpu.make_async_copy(
                    rows_f.at[sb], acc_sh.at[dst_v.at[g]], ssems.at[sb]).wait()

            @pl.when(g + 2 < IG)
            def _():
                gb2 = lax.rem(g + 2, NBG)
                pltpu.async_copy(
                    h_hbm.at[src_v.at[g + 2]], rows_bf.at[gb2], gsems.at[gb2])

            pltpu.make_async_copy(
                h_hbm.at[src_v.at[g]], rows_bf.at[gb], gsems.at[gb]).wait()

            def _scale(t, cc):
                vblock = vals_v[g, pl.ds(t * 16, 16)]
                for k in range(16):
                    i = t * 16 + k
                    vv = jnp.full((16,), vblock[k], jnp.float32)
                    for j in range(D // 32):
                        x = rows_bf[gb, i, pl.ds(j * 16, 16)]
                        lo = lax.bitcast_convert_type(
                            lax.shift_left(x, 16), jnp.float32)
                        hi = lax.bitcast_convert_type(
                            lax.bitwise_and(x, hi_mask), jnp.float32)
                        rows_f[sb, i, pl.ds(j * 32, 16)] = lo * vv
                        rows_f[sb, i, pl.ds(j * 32 + 16, 16)] = hi * vv
                return cc

            lax.fori_loop(0, CH // 16, _scale, 0)
            pltpu.async_copy(
                rows_f.at[sb], acc_sh.at[dst_v.at[g]], ssems.at[sb], add=True)
            return cc2

        lax.fori_loop(0, IG, _chunk, 0)
        for gl in range(IG - NBS, IG):
            b = gl % NBS
            pltpu.make_async_copy(
                rows_f.at[b], acc_sh.at[dst_v.at[0]], ssems.at[b]).wait()
        return carry

    lax.fori_loop(0, IGN, _outer, 0)
    plsc.subcore_barrier()

    @pl.when(c == 0)
    def _():
        pltpu.sync_copy(acc_sh.at[pl.ds(row0, RS)], out0_hbm.at[pl.ds(row0, RS)])

    @pl.when(c == 1)
    def _():
        pltpu.sync_copy(acc_sh.at[pl.ds(row0, RS)], out1_hbm.at[pl.ds(row0, RS)])


_spmm = functools.partial(
    pl.kernel,
    mesh=plsc.VectorSubcoreMesh(core_axis_name="c", subcore_axis_name="s"),
    compiler_params=pltpu.CompilerParams(
        needs_layout_passes=False, use_tc_tiling_on_sc=False),
    out_type=[jax.ShapeDtypeStruct((N_PAD, D), jnp.float32),
              jax.ShapeDtypeStruct((N_PAD, D), jnp.float32)],
    scratch_types=[
        pltpu.VMEM((IG, CH), jnp.int32),
        pltpu.VMEM((IG, CH), jnp.int32),
        pltpu.VMEM((IG, CH), jnp.float32),
        pltpu.VMEM((NBG, CH, DW), jnp.int32),
        pltpu.VMEM((NBS, CH, D), jnp.float32),
        pltpu.VMEM((ZR, D), jnp.float32),
        pltpu.VMEM_SHARED((N_PAD, D), jnp.float32),
        pltpu.SemaphoreType.DMA((NBG,)),
        pltpu.SemaphoreType.DMA((NBS,)),
    ],
)(_spmm_body)


BN = 2000


def _dense_body(coef_ref, p0_ref, p1_ref, h0_ref, w_ref, out_ref):
    alpha = coef_ref[0]
    c1 = coef_ref[1]
    c2 = coef_ref[2]
    support = (1.0 - alpha) * (p0_ref[...] + p1_ref[...]) + alpha * h0_ref[...]
    out_ref[...] = c1 * support + c2 * jnp.dot(
        support, w_ref[...], preferred_element_type=jnp.float32)


def kernel(edge_index, A_vals, H, H0, weight, alpha, lamda, l):
    src = edge_index[0].astype(jnp.int32)
    dst = edge_index[1].astype(jnp.int32)
    vals = A_vals.astype(jnp.float32)
    pad = E_PAD - E
    src_p = jnp.concatenate([src, jnp.zeros((pad,), jnp.int32)])
    dst_p = jnp.concatenate([dst, jnp.zeros((pad,), jnp.int32)])
    vals_p = jnp.concatenate([vals, jnp.zeros((pad,), jnp.float32)])
    src3 = src_p.reshape(NW, IGN, IG, CH)
    dst3 = dst_p.reshape(NW, IGN, IG, CH)
    vals3 = vals_p.reshape(NW, IGN, IG, CH)

    h_pre = H[:, jnp.asarray(_COL_SRC)].astype(jnp.bfloat16)
    h_packed = lax.bitcast_convert_type(h_pre.reshape(N, DW, 2), jnp.int32)

    part0, part1 = _spmm(src3, dst3, vals3, h_packed)

    beta = math.log(1 / 4 + 1.0)
    unit = jnp.asarray((lamda * l) // (l * lamda)).astype(jnp.float32)
    c1 = (1.0 - beta) * unit
    c2 = beta * unit
    coefs = jnp.stack([alpha.astype(jnp.float32), c1, c2])

    return pl.pallas_call(
        _dense_body,
        grid=(N // BN,),
        in_specs=[
            pl.BlockSpec(memory_space=pltpu.SMEM),
            pl.BlockSpec((BN, D), lambda i: (i, 0)),
            pl.BlockSpec((BN, D), lambda i: (i, 0)),
            pl.BlockSpec((BN, D), lambda i: (i, 0)),
            pl.BlockSpec((D, D), lambda i: (0, 0)),
        ],
        out_specs=pl.BlockSpec((BN, D), lambda i: (i, 0)),
        out_shape=jax.ShapeDtypeStruct((N, D), jnp.float32),
    )(coefs, part0[:N], part1[:N], H0, weight)


# speedup vs baseline: 1.2816x; 1.2816x over previous
import functools
import math

import jax
import jax.numpy as jnp
from jax import lax
from jax.experimental import pallas as pl
from jax.experimental.pallas import tpu as pltpu
from jax.experimental.pallas import tpu_sc as plsc

N = 10000
E = 320000
D = 128

NC = 2
NS = 16
NW = NC * NS

CH = 96
IG = 21
IGN = 5
NCHUNK = IG * IGN
EPW = NCHUNK * CH
E_PAD = NW * EPW
NBUF = 3

N_PAD = 10240
RS = N_PAD // NS
ZR = 16


def _spmm_body(src_hbm, dst_hbm, vals_hbm, h_hbm, out0_hbm, out1_hbm,
               src_v, dst_v, vals_v, rows_v, zbuf, acc_sh, gsems, ssems):
    c = lax.axis_index("c")
    s = lax.axis_index("s")
    w = c * NS + s
    row0 = s * RS

    def _zero_row(i, carry):
        for j in range(D // 16):
            zbuf[i, pl.ds(j * 16, 16)] = jnp.zeros((16,), jnp.float32)
        return carry

    lax.fori_loop(0, ZR, _zero_row, 0)
    for k in range(RS // ZR):
        pltpu.sync_copy(zbuf, acc_sh.at[pl.ds(row0 + k * ZR, ZR)])
    plsc.subcore_barrier()

    def _outer(og, carry):
        pltpu.sync_copy(src_hbm.at[w, og], src_v)
        pltpu.sync_copy(dst_hbm.at[w, og], dst_v)
        pltpu.sync_copy(vals_hbm.at[w, og], vals_v)

        pltpu.async_copy(h_hbm.at[src_v.at[0]], rows_v.at[0], gsems.at[0])

        def _chunk(g, cc2):
            for b in range(NBUF):

                @pl.when(lax.rem(g, NBUF) == b)
                def _():
                    bn = (b + 1) % NBUF

                    @pl.when(g >= NBUF - 1)
                    def _():
                        pltpu.make_async_copy(
                            rows_v.at[bn], acc_sh.at[dst_v.at[g]],
                            ssems.at[bn]).wait()

                    @pl.when(g + 1 < IG)
                    def _():
                        pltpu.async_copy(
                            h_hbm.at[src_v.at[g + 1]], rows_v.at[bn],
                            gsems.at[bn])

                    pltpu.make_async_copy(
                        h_hbm.at[src_v.at[g]], rows_v.at[b], gsems.at[b]).wait()

                    def _scale(t, cc):
                        vblock = vals_v[g, pl.ds(t * 16, 16)]
                        for k in range(16):
                            vv = jnp.full((16,), vblock[k], jnp.float32)
                            for j in range(D // 16):
                                rows_v[b, t * 16 + k, pl.ds(j * 16, 16)] = (
                                    rows_v[b, t * 16 + k, pl.ds(j * 16, 16)] * vv)
                        return cc

                    lax.fori_loop(0, CH // 16, _scale, 0)
                    pltpu.async_copy(
                        rows_v.at[b], acc_sh.at[dst_v.at[g]], ssems.at[b],
                        add=True)
            return cc2

        lax.fori_loop(0, IG, _chunk, 0)
        for b in ((IG - 2) % NBUF, (IG - 1) % NBUF):
            pltpu.make_async_copy(
                rows_v.at[b], acc_sh.at[dst_v.at[0]], ssems.at[b]).wait()
        return carry

    lax.fori_loop(0, IGN, _outer, 0)
    plsc.subcore_barrier()

    @pl.when(c == 0)
    def _():
        pltpu.sync_copy(acc_sh.at[pl.ds(row0, RS)], out0_hbm.at[pl.ds(row0, RS)])

    @pl.when(c == 1)
    def _():
        pltpu.sync_copy(acc_sh.at[pl.ds(row0, RS)], out1_hbm.at[pl.ds(row0, RS)])


_spmm = functools.partial(
    pl.kernel,
    mesh=plsc.VectorSubcoreMesh(core_axis_name="c", subcore_axis_name="s"),
    out_type=[jax.ShapeDtypeStruct((N_PAD, D), jnp.float32),
              jax.ShapeDtypeStruct((N_PAD, D), jnp.float32)],
    scratch_types=[
        pltpu.VMEM((IG, CH), jnp.int32),
        pltpu.VMEM((IG, CH), jnp.int32),
        pltpu.VMEM((IG, CH), jnp.float32),
        pltpu.VMEM((NBUF, CH, D), jnp.float32),
        pltpu.VMEM((ZR, D), jnp.float32),
        pltpu.VMEM_SHARED((N_PAD, D), jnp.float32),
        pltpu.SemaphoreType.DMA((NBUF,)),
        pltpu.SemaphoreType.DMA((NBUF,)),
    ],
)(_spmm_body)


BN = 2000


def _dense_body(coef_ref, p0_ref, p1_ref, h0_ref, w_ref, out_ref):
    alpha = coef_ref[0]
    c1 = coef_ref[1]
    c2 = coef_ref[2]
    support = (1.0 - alpha) * (p0_ref[...] + p1_ref[...]) + alpha * h0_ref[...]
    out_ref[...] = c1 * support + c2 * jnp.dot(
        support, w_ref[...], preferred_element_type=jnp.float32)


def kernel(edge_index, A_vals, H, H0, weight, alpha, lamda, l):
    src = edge_index[0].astype(jnp.int32)
    dst = edge_index[1].astype(jnp.int32)
    vals = A_vals.astype(jnp.float32)
    pad = E_PAD - E
    src_p = jnp.concatenate([src, jnp.zeros((pad,), jnp.int32)])
    dst_p = jnp.concatenate([dst, jnp.zeros((pad,), jnp.int32)])
    vals_p = jnp.concatenate([vals, jnp.zeros((pad,), jnp.float32)])
    src3 = src_p.reshape(NW, IGN, IG, CH)
    dst3 = dst_p.reshape(NW, IGN, IG, CH)
    vals3 = vals_p.reshape(NW, IGN, IG, CH)

    part0, part1 = _spmm(src3, dst3, vals3, H)

    beta = math.log(1 / 4 + 1.0)
    unit = jnp.asarray((lamda * l) // (l * lamda)).astype(jnp.float32)
    c1 = (1.0 - beta) * unit
    c2 = beta * unit
    coefs = jnp.stack([alpha.astype(jnp.float32), c1, c2])

    p0 = part0[:N]
    p1 = part1[:N]

    return pl.pallas_call(
        _dense_body,
        grid=(N // BN,),
        in_specs=[
            pl.BlockSpec(memory_space=pltpu.SMEM),
            pl.BlockSpec((BN, D), lambda i: (i, 0)),
            pl.BlockSpec((BN, D), lambda i: (i, 0)),
            pl.BlockSpec((BN, D), lambda i: (i, 0)),
            pl.BlockSpec((D, D), lambda i: (0, 0)),
        ],
        out_specs=pl.BlockSpec((BN, D), lambda i: (i, 0)),
        out_shape=jax.ShapeDtypeStruct((N, D), jnp.float32),
    )(coefs, p0, p1, H0, weight)


# speedup vs baseline: 2.1735x; 1.6959x over previous
import functools
import math

import jax
import jax.numpy as jnp
from jax import lax
from jax.experimental import pallas as pl
from jax.experimental.pallas import tpu as pltpu
from jax.experimental.pallas import tpu_sc as plsc

N = 10000
E = 320000
D = 128

NC = 2
NS = 16
NW = NC * NS

CH = 80
IG = 25
IGN = 5
NCHUNK = IG * IGN
EPW = NCHUNK * CH
NBUF = 3

N_PAD = 10240
RS = N_PAD // NS
ZR = 16


def _spmm_body(src_hbm, dst_hbm, vals_hbm, h_hbm, out0_hbm, out1_hbm,
               src_v, dst_v, vals_v, rows_v, zbuf, acc_sh, gsems, ssems):
    c = lax.axis_index("c")
    s = lax.axis_index("s")
    w = c * NS + s
    row0 = s * RS

    def _zero_row(i, carry):
        for j in range(D // 16):
            zbuf[i, pl.ds(j * 16, 16)] = jnp.zeros((16,), jnp.float32)
        return carry

    lax.fori_loop(0, ZR, _zero_row, 0)
    for k in range(RS // ZR):
        pltpu.sync_copy(zbuf, acc_sh.at[pl.ds(row0 + k * ZR, ZR)])
    plsc.subcore_barrier()

    def _outer(og, carry):
        pltpu.sync_copy(src_hbm.at[w, og], src_v)
        pltpu.sync_copy(dst_hbm.at[w, og], dst_v)
        pltpu.sync_copy(vals_hbm.at[w, og], vals_v)

        pltpu.async_copy(h_hbm.at[src_v.at[0]], rows_v.at[0], gsems.at[0])

        def _chunk(g, cc2):
            for b in range(NBUF):

                @pl.when(lax.rem(g, NBUF) == b)
                def _():
                    bn = (b + 1) % NBUF

                    @pl.when(g >= NBUF - 1)
                    def _():
                        pltpu.make_async_copy(
                            rows_v.at[bn], acc_sh.at[dst_v.at[g]],
                            ssems.at[bn]).wait()

                    @pl.when(g + 1 < IG)
                    def _():
                        pltpu.async_copy(
                            h_hbm.at[src_v.at[g + 1]], rows_v.at[bn],
                            gsems.at[bn])

                    pltpu.make_async_copy(
                        h_hbm.at[src_v.at[g]], rows_v.at[b], gsems.at[b]).wait()

                    def _scale(t, cc):
                        vblock = vals_v[g, pl.ds(t * 16, 16)]
                        for k in range(16):
                            vv = jnp.full((16,), vblock[k], jnp.float32)
                            for j in range(D // 16):
                                rows_v[b, t * 16 + k, pl.ds(j * 16, 16)] = (
                                    rows_v[b, t * 16 + k, pl.ds(j * 16, 16)] * vv)
                        return cc

                    lax.fori_loop(0, CH // 16, _scale, 0)
                    pltpu.async_copy(
                        rows_v.at[b], acc_sh.at[dst_v.at[g]], ssems.at[b],
                        add=True)
            return cc2

        lax.fori_loop(0, IG, _chunk, 0)
        for b in ((IG - 2) % NBUF, (IG - 1) % NBUF):
            pltpu.make_async_copy(
                rows_v.at[b], acc_sh.at[dst_v.at[0]], ssems.at[b]).wait()
        return carry

    lax.fori_loop(0, IGN, _outer, 0)
    plsc.subcore_barrier()

    @pl.when(c == 0)
    def _():
        pltpu.sync_copy(acc_sh.at[pl.ds(row0, RS)], out0_hbm.at[pl.ds(row0, RS)])

    @pl.when(c == 1)
    def _():
        pltpu.sync_copy(acc_sh.at[pl.ds(row0, RS)], out1_hbm.at[pl.ds(row0, RS)])


_spmm = functools.partial(
    pl.kernel,
    mesh=plsc.VectorSubcoreMesh(core_axis_name="c", subcore_axis_name="s"),
    out_type=[jax.ShapeDtypeStruct((N_PAD, D), jnp.float32),
              jax.ShapeDtypeStruct((N_PAD, D), jnp.float32)],
    scratch_types=[
        pltpu.VMEM((IG, CH), jnp.int32),
        pltpu.VMEM((IG, CH), jnp.int32),
        pltpu.VMEM((IG, CH), jnp.float32),
        pltpu.VMEM((NBUF, CH, D), jnp.float32),
        pltpu.VMEM((ZR, D), jnp.float32),
        pltpu.VMEM_SHARED((N_PAD, D), jnp.float32),
        pltpu.SemaphoreType.DMA((NBUF,)),
        pltpu.SemaphoreType.DMA((NBUF,)),
    ],
)(_spmm_body)


BN = 2000


def _dense_body(coef_ref, p0_ref, p1_ref, h0_ref, w_ref, out_ref):
    alpha = coef_ref[0]
    c1 = coef_ref[1]
    c2 = coef_ref[2]
    support = (1.0 - alpha) * (p0_ref[...] + p1_ref[...]) + alpha * h0_ref[...]
    out_ref[...] = c1 * support + c2 * jnp.dot(
        support, w_ref[...], preferred_element_type=jnp.float32)


def kernel(edge_index, A_vals, H, H0, weight, alpha, lamda, l):
    src = edge_index[0].astype(jnp.int32)
    dst = edge_index[1].astype(jnp.int32)
    vals = A_vals.astype(jnp.float32)
    src3 = src.reshape(NW, IGN, IG, CH)
    dst3 = dst.reshape(NW, IGN, IG, CH)
    vals3 = vals.reshape(NW, IGN, IG, CH)

    part0, part1 = _spmm(src3, dst3, vals3, H)

    beta = math.log(1 / 4 + 1.0)
    unit = jnp.asarray((lamda * l) // (l * lamda)).astype(jnp.float32)
    c1 = (1.0 - beta) * unit
    c2 = beta * unit
    coefs = jnp.stack([alpha.astype(jnp.float32), c1, c2])

    p0 = part0[:N]
    p1 = part1[:N]

    return pl.pallas_call(
        _dense_body,
        grid=(N // BN,),
        in_specs=[
            pl.BlockSpec(memory_space=pltpu.SMEM),
            pl.BlockSpec((BN, D), lambda i: (i, 0)),
            pl.BlockSpec((BN, D), lambda i: (i, 0)),
            pl.BlockSpec((BN, D), lambda i: (i, 0)),
            pl.BlockSpec((D, D), lambda i: (0, 0)),
        ],
        out_specs=pl.BlockSpec((BN, D), lambda i: (i, 0)),
        out_shape=jax.ShapeDtypeStruct((N, D), jnp.float32),
    )(coefs, p0, p1, H0, weight)
